# Initial kernel scaffold; baseline (speedup 1.0000x reference)
import jax
import jax.numpy as jnp
from jax.experimental import pallas as pl


def kernel(x, projector, data, labels):
    raise NotImplementedError("write your pallas kernel here")


# speedup vs baseline: 1.7179x; 1.7179x over previous
import jax
import jax.numpy as jnp
from jax.experimental import pallas as pl

_PROJ = 30
_K = 15
_NCLS = 10
_C = 2048
_BQ = 512
_BX = 256


def _proj_body(x_ref, p_ref, q_ref):
    xb = x_ref[...]
    m = jnp.mean(xb, axis=1, keepdims=True)
    xc = xb - m
    nrm = jnp.sqrt(jnp.sum(xc * xc, axis=1, keepdims=True))
    xn = xc / nrm
    q_ref[...] = jnp.dot(xn, p_ref[...], preferred_element_type=jnp.float32)


def _knn_body(qt_ref, data_ref, labt_ref, out_ref, *, n_chunks):
    qt = qt_ref[...]
    b2 = jnp.sum(qt * qt, axis=0, keepdims=True)
    bigi = jnp.int32(2**31 - 1)

    def dist_chunk(j):
        dc = data_ref[pl.ds(j * _C, _C), :]
        a2 = jnp.sum(dc * dc, axis=1, keepdims=True)
        ab = jnp.dot(dc, qt, preferred_element_type=jnp.float32)
        sq = jnp.maximum((a2 + b2) - 2.0 * ab, 1e-12)
        return jnp.sqrt(sq)

    def phase1(j, carry):
        top_d, top_i = carry
        d = dist_chunk(j)
        gi = j * _C + jax.lax.broadcasted_iota(jnp.int32, (_C, _BQ), 0)
        wd = jnp.concatenate([top_d, d], axis=0)
        wi = jnp.concatenate([top_i, gi], axis=0)
        nd, ni = [], []
        for _ in range(_K):
            m = jnp.min(wd, axis=0, keepdims=True)
            eq = wd == m
            c = jnp.min(jnp.where(eq, wi, bigi), axis=0, keepdims=True)
            sel = eq & (wi == c)
            wd = jnp.where(sel, jnp.float32(jnp.inf), wd)
            nd.append(m)
            ni.append(c)
        nd.append(jnp.full((1, _BQ), jnp.inf, jnp.float32))
        ni.append(jnp.full((1, _BQ), bigi, jnp.int32))
        return jnp.concatenate(nd, axis=0), jnp.concatenate(ni, axis=0)

    top_d0 = jnp.full((_K + 1, _BQ), jnp.inf, jnp.float32)
    top_i0 = jnp.full((_K + 1, _BQ), bigi, jnp.int32)
    top_d, top_i = jax.lax.fori_loop(0, n_chunks, phase1, (top_d0, top_i0))
    t = top_d[_K - 1:_K, :]
    i14 = top_i[_K - 1:_K, :]

    def phase2(j, s):
        d = dist_chunk(j)
        gi = j * _C + jax.lax.broadcasted_iota(jnp.int32, (_C, _BQ), 0)
        mask = (d < t) | ((d == t) & (gi <= i14))
        w = jnp.where(mask, jnp.exp(-d), 0.0)
        lt = labt_ref[:, pl.ds(j * _C, _C)]
        return s + jnp.dot(lt, w, preferred_element_type=jnp.float32)

    s = jax.lax.fori_loop(0, n_chunks, phase2,
                          jnp.zeros((_NCLS, _BQ), jnp.float32))
    out_ref[...] = jnp.log(s)


def kernel(x, projector, data, labels):
    b = x.shape[0]
    x2 = x.reshape(b, -1)
    q = pl.pallas_call(
        _proj_body,
        grid=(b // _BX,),
        in_specs=[
            pl.BlockSpec((_BX, x2.shape[1]), lambda i: (i, 0)),
            pl.BlockSpec((x2.shape[1], _PROJ), lambda i: (0, 0)),
        ],
        out_specs=pl.BlockSpec((_BX, _PROJ), lambda i: (i, 0)),
        out_shape=jax.ShapeDtypeStruct((b, _PROJ), jnp.float32),
    )(x2, projector[:, :_PROJ])

    qt = q.T
    d0 = data[0]
    n = d0.shape[0]
    n_pad = ((n + _C - 1) // _C) * _C
    n_chunks = n_pad // _C
    dpad = jnp.concatenate(
        [d0, jnp.full((n_pad - n, _PROJ), 1e6, jnp.float32)], axis=0)
    labt = jnp.concatenate(
        [labels.T, jnp.zeros((_NCLS, n_pad - n), jnp.float32)], axis=1)

    import functools
    out = pl.pallas_call(
        functools.partial(_knn_body, n_chunks=n_chunks),
        grid=(b // _BQ,),
        in_specs=[
            pl.BlockSpec((_PROJ, _BQ), lambda i: (0, i)),
            pl.BlockSpec((n_pad, _PROJ), lambda i: (0, 0)),
            pl.BlockSpec((_NCLS, n_pad), lambda i: (0, 0)),
        ],
        out_specs=pl.BlockSpec((_NCLS, _BQ), lambda i: (0, i)),
        out_shape=jax.ShapeDtypeStruct((_NCLS, b), jnp.float32),
    )(qt, dpad, labt)
    return out.T


# speedup vs baseline: 2.0397x; 1.1873x over previous
import functools

import jax
import jax.numpy as jnp
from jax import lax
from jax.experimental import pallas as pl
from jax.experimental.pallas import tpu as pltpu
from jax.experimental.pallas import tpu_sc as plsc

_PROJ = 30
_K = 15
_NCLS = 10
_C = 2048
_SEG = 128
_BQ = 512
_BX = 256
_NSC = 32
_GW = 128


def _proj_body(x_ref, p_ref, q_ref):
    xb = x_ref[...]
    m = jnp.mean(xb, axis=1, keepdims=True)
    xc = xb - m
    nrm = jnp.sqrt(jnp.sum(xc * xc, axis=1, keepdims=True))
    xn = xc / nrm
    q_ref[...] = jnp.dot(xn, p_ref[...], preferred_element_type=jnp.float32)


def _knn_body(qt_ref, data_ref, outd_ref, outi_ref, *, n_chunks, n_real):
    qt = qt_ref[...]
    b2 = jnp.sum(qt * qt, axis=0, keepdims=True)
    bigi = jnp.int32(2**31 - 1)
    inf = jnp.float32(jnp.inf)
    riota = lax.broadcasted_iota(jnp.int32, (_K + 1, _BQ), 0)

    def sq_chunk(j):
        dc = data_ref[pl.ds(j * _C, _C), :]
        a2 = jnp.sum(dc * dc, axis=1, keepdims=True)
        ab = jnp.dot(dc, qt, preferred_element_type=jnp.float32)
        return jnp.maximum((a2 + b2) - 2.0 * ab, 1e-12)

    def insert(top_d, top_i, m, c):
        lt = (top_d < m) | ((top_d == m) & (top_i < c))
        pos = jnp.sum(lt.astype(jnp.int32), axis=0, keepdims=True)
        sh_d = jnp.roll(top_d, 1, axis=0)
        sh_i = jnp.roll(top_i, 1, axis=0)
        at = riota == pos
        nd = jnp.where(lt, top_d, jnp.where(at, m, sh_d))
        ni = jnp.where(lt, top_i, jnp.where(at, c, sh_i))
        keep = m == inf
        nd = jnp.where(keep, top_d, nd)
        ni = jnp.where(keep, top_i, ni)
        nd = jnp.where(riota == _K, inf, nd)
        ni = jnp.where(riota == _K, bigi, ni)
        return nd, ni

    n_seg = _C // _SEG

    def merge_chunk(j, carry):
        top_d, top_i = carry
        sq = sq_chunk(j)
        gi = j * _C + lax.broadcasted_iota(jnp.int32, (_C, _BQ), 0)
        t = top_d[_K - 1:_K, :]
        dm = jnp.where(sq <= t * t * 1.000001, sq, inf)

        def cond(state):
            _, left, _, _ = state
            return left

        def sweep(state):
            dm, _, top_d, top_i = state
            segs, mins = [], []
            for s in range(n_seg):
                ds = lax.slice(dm, (s * _SEG, 0), ((s + 1) * _SEG, _BQ))
                gs = lax.slice(gi, (s * _SEG, 0), ((s + 1) * _SEG, _BQ))
                m = jnp.min(ds, axis=0, keepdims=True)
                eq = ds == m
                c = jnp.min(jnp.where(eq, gs, bigi), axis=0, keepdims=True)
                nds = jnp.where(eq & (gs == c), inf, ds)
                segs.append(nds)
                mins.append(jnp.min(nds))
                top_d, top_i = insert(top_d, top_i, jnp.sqrt(m), c)
            left = functools.reduce(jnp.minimum, mins) < inf
            return jnp.concatenate(segs, axis=0), left, top_d, top_i

        _, _, top_d, top_i = lax.while_loop(
            cond, sweep, (dm, jnp.min(dm) < inf, top_d, top_i))
        return top_d, top_i

    top_d0 = jnp.full((_K + 1, _BQ), inf, jnp.float32)
    top_i0 = jnp.full((_K + 1, _BQ), bigi, jnp.int32)
    top_d, top_i = lax.fori_loop(0, n_chunks, merge_chunk, (top_d0, top_i0))
    outd_ref[...] = jnp.minimum(top_d, 1e30).T
    outi_ref[...] = jnp.minimum(top_i, n_real - 1).T


def _cls_body(lab_ref, cls_ref):
    ids = lax.broadcasted_iota(jnp.int32, lab_ref.shape, 1).astype(jnp.float32)
    cls_ref[...] = jnp.sum(lab_ref[...] * ids, axis=1).astype(jnp.int32)


def _sc_body(topd_hbm, topi_hbm, cls_hbm, out_hbm, d_v, i_v, cls_v, o_v):
    qpw = d_v.shape[0]
    wid = lax.axis_index("s") * 2 + lax.axis_index("c")
    base = wid * qpw
    pltpu.sync_copy(topd_hbm.at[pl.ds(base, qpw)], d_v)
    pltpu.sync_copy(topi_hbm.at[pl.ds(base, qpw)], i_v)
    pltpu.sync_copy(cls_hbm, cls_v)
    lane = lax.iota(jnp.int32, 16)

    def per_query(q, _):
        wv = jnp.exp(-d_v[q, :])
        cv = plsc.load_gather(cls_v, [i_v[q, :]])
        acc = jnp.zeros((16,), jnp.float32)
        for k in range(_K):
            acc = acc + jnp.where(lane == cv[k], wv[k], 0.0)
        o_v[q, :] = acc
        return 0

    lax.fori_loop(0, qpw, per_query, 0)
    pltpu.sync_copy(o_v, out_hbm.at[pl.ds(base, qpw)])


def _log_body(s_ref, o_ref):
    o_ref[...] = jnp.log(s_ref[:, :_NCLS])


def kernel(x, projector, data, labels):
    b = x.shape[0]
    x2 = x.reshape(b, -1)
    q = pl.pallas_call(
        _proj_body,
        grid=(b // _BX,),
        in_specs=[
            pl.BlockSpec((_BX, x2.shape[1]), lambda i: (i, 0)),
            pl.BlockSpec((x2.shape[1], _PROJ), lambda i: (0, 0)),
        ],
        out_specs=pl.BlockSpec((_BX, _PROJ), lambda i: (i, 0)),
        out_shape=jax.ShapeDtypeStruct((b, _PROJ), jnp.float32),
    )(x2, projector[:, :_PROJ])

    qt = q.T
    d0 = data[0]
    n = d0.shape[0]
    n_pad = ((n + _C - 1) // _C) * _C
    n_chunks = n_pad // _C
    dpad = jnp.concatenate(
        [d0, jnp.full((n_pad - n, _PROJ), 1e6, jnp.float32)], axis=0)

    td, ti = pl.pallas_call(
        functools.partial(_knn_body, n_chunks=n_chunks, n_real=n),
        grid=(b // _BQ,),
        in_specs=[
            pl.BlockSpec((_PROJ, _BQ), lambda i: (0, i)),
            pl.BlockSpec((n_pad, _PROJ), lambda i: (0, 0)),
        ],
        out_specs=[
            pl.BlockSpec((_BQ, 16), lambda i: (i, 0)),
            pl.BlockSpec((_BQ, 16), lambda i: (i, 0)),
        ],
        out_shape=[
            jax.ShapeDtypeStruct((b, 16), jnp.float32),
            jax.ShapeDtypeStruct((b, 16), jnp.int32),
        ],
    )(qt, dpad)

    qpw = b // _NSC
    cls = pl.pallas_call(
        _cls_body,
        grid=(1,),
        in_specs=[pl.BlockSpec((n, _NCLS), lambda i: (0, 0))],
        out_specs=pl.BlockSpec((n,), lambda i: (0,)),
        out_shape=jax.ShapeDtypeStruct((n,), jnp.int32),
    )(labels)

    mesh = plsc.VectorSubcoreMesh(core_axis_name="c", subcore_axis_name="s")
    s16 = pl.kernel(
        _sc_body,
        out_type=jax.ShapeDtypeStruct((b, 16), jnp.float32),
        mesh=mesh,
        scratch_types=[
            pltpu.VMEM((qpw, 16), jnp.float32),
            pltpu.VMEM((qpw, 16), jnp.int32),
            pltpu.VMEM((n,), jnp.int32),
            pltpu.VMEM((qpw, 16), jnp.float32),
        ],
        compiler_params=pltpu.CompilerParams(needs_layout_passes=False),
    )(td, ti, cls)

    out = pl.pallas_call(
        _log_body,
        grid=(b // _BQ,),
        in_specs=[pl.BlockSpec((_BQ, 16), lambda i: (i, 0))],
        out_specs=pl.BlockSpec((_BQ, _NCLS), lambda i: (i, 0)),
        out_shape=jax.ShapeDtypeStruct((b, _NCLS), jnp.float32),
    )(s16)
    return out


# speedup vs baseline: 6.9593x; 3.4118x over previous
import functools

import jax
import jax.numpy as jnp
from jax import lax
from jax.experimental import pallas as pl
from jax.experimental.pallas import tpu as pltpu
from jax.experimental.pallas import tpu_sc as plsc

_PROJ = 30
_K = 15
_NCLS = 10
_C = 2048
_SEG = 128
_BQ = 512
_BX = 256
_NSC = 32
_GW = 128


def _proj_body(x_ref, p_ref, q_ref):
    xb = x_ref[...]
    m = jnp.mean(xb, axis=1, keepdims=True)
    xc = xb - m
    nrm = jnp.sqrt(jnp.sum(xc * xc, axis=1, keepdims=True))
    xn = xc / nrm
    q_ref[...] = jnp.dot(xn, p_ref[...], preferred_element_type=jnp.float32)


def _knn_body(qt_ref, data_ref, outd_ref, outi_ref, *, n_chunks, n_real):
    qt = qt_ref[...]
    b2 = jnp.sum(qt * qt, axis=0, keepdims=True)
    bigi = jnp.int32(2**31 - 1)
    inf = jnp.float32(jnp.inf)
    riota = lax.broadcasted_iota(jnp.int32, (_K + 1, _BQ), 0)

    def sq_chunk(j):
        dc = data_ref[pl.ds(j * _C, _C), :]
        a2 = jnp.sum(dc * dc, axis=1, keepdims=True)
        ab = jnp.dot(dc, qt, preferred_element_type=jnp.float32)
        return jnp.maximum((a2 + b2) - 2.0 * ab, 1e-12)

    def insert(top_d, top_i, m, c):
        lt = (top_d < m) | ((top_d == m) & (top_i < c))
        pos = jnp.sum(lt.astype(jnp.int32), axis=0, keepdims=True)
        sh_d = jnp.roll(top_d, 1, axis=0)
        sh_i = jnp.roll(top_i, 1, axis=0)
        at = riota == pos
        nd = jnp.where(lt, top_d, jnp.where(at, m, sh_d))
        ni = jnp.where(lt, top_i, jnp.where(at, c, sh_i))
        keep = m == inf
        nd = jnp.where(keep, top_d, nd)
        ni = jnp.where(keep, top_i, ni)
        nd = jnp.where(riota == _K, inf, nd)
        ni = jnp.where(riota == _K, bigi, ni)
        return nd, ni

    n_seg = _C // _SEG

    def merge_chunk(j, carry):
        top_d, top_i = carry
        sq = sq_chunk(j)
        gi = j * _C + lax.broadcasted_iota(jnp.int32, (_C, _BQ), 0)
        t = top_d[_K - 1:_K, :]
        dm = jnp.where(sq <= t * t * 1.000001, sq, inf)

        def cond(state):
            dm, _, _ = state
            return jnp.min(dm) < inf

        def sweep(state):
            dm, top_d, top_i = state
            segs = []
            for s in range(n_seg):
                ds = lax.slice(dm, (s * _SEG, 0), ((s + 1) * _SEG, _BQ))
                gs = lax.slice(gi, (s * _SEG, 0), ((s + 1) * _SEG, _BQ))
                m = jnp.min(ds, axis=0, keepdims=True)
                eq = ds == m
                c = jnp.min(jnp.where(eq, gs, bigi), axis=0, keepdims=True)
                segs.append(jnp.where(eq & (gs == c), inf, ds))
                top_d, top_i = insert(top_d, top_i, jnp.sqrt(m), c)
            dm = jnp.concatenate(segs, axis=0)
            t = top_d[_K - 1:_K, :]
            dm = jnp.where(dm <= t * t * 1.000001, dm, inf)
            return dm, top_d, top_i

        dm, top_d, top_i = lax.while_loop(cond, sweep, (dm, top_d, top_i))
        return top_d, top_i

    top_d0 = jnp.full((_K + 1, _BQ), inf, jnp.float32)
    top_i0 = jnp.full((_K + 1, _BQ), bigi, jnp.int32)
    top_d, top_i = lax.fori_loop(0, n_chunks, merge_chunk, (top_d0, top_i0))
    outd_ref[...] = jnp.minimum(top_d, 1e30).T
    outi_ref[...] = jnp.minimum(top_i, n_real - 1).T


def _cls_body(lab_ref, cls_ref):
    ids = lax.broadcasted_iota(jnp.int32, lab_ref.shape, 1).astype(jnp.float32)
    cls_ref[...] = jnp.sum(lab_ref[...] * ids, axis=1).astype(jnp.int32)


def _sc_body(topd_hbm, topi_hbm, cls_hbm, out_hbm, d_v, i_v, cls_v, o_v):
    qpw = d_v.shape[0]
    wid = lax.axis_index("s") * 2 + lax.axis_index("c")
    base = wid * qpw
    pltpu.sync_copy(topd_hbm.at[pl.ds(base, qpw)], d_v)
    pltpu.sync_copy(topi_hbm.at[pl.ds(base, qpw)], i_v)
    pltpu.sync_copy(cls_hbm, cls_v)
    lane = lax.iota(jnp.int32, 16)

    def per_query(q, _):
        wv = jnp.exp(-d_v[q, :])
        cv = plsc.load_gather(cls_v, [i_v[q, :]])
        acc = jnp.zeros((16,), jnp.float32)
        for k in range(_K):
            acc = acc + jnp.where(lane == cv[k], wv[k], 0.0)
        o_v[q, :] = acc
        return 0

    lax.fori_loop(0, qpw, per_query, 0)
    pltpu.sync_copy(o_v, out_hbm.at[pl.ds(base, qpw)])


def _log_body(s_ref, o_ref):
    o_ref[...] = jnp.log(s_ref[:, :_NCLS])


def kernel(x, projector, data, labels):
    b = x.shape[0]
    x2 = x.reshape(b, -1)
    q = pl.pallas_call(
        _proj_body,
        grid=(b // _BX,),
        in_specs=[
            pl.BlockSpec((_BX, x2.shape[1]), lambda i: (i, 0)),
            pl.BlockSpec((x2.shape[1], _PROJ), lambda i: (0, 0)),
        ],
        out_specs=pl.BlockSpec((_BX, _PROJ), lambda i: (i, 0)),
        out_shape=jax.ShapeDtypeStruct((b, _PROJ), jnp.float32),
    )(x2, projector[:, :_PROJ])

    qt = q.T
    d0 = data[0]
    n = d0.shape[0]
    n_pad = ((n + _C - 1) // _C) * _C
    n_chunks = n_pad // _C
    dpad = jnp.concatenate(
        [d0, jnp.full((n_pad - n, _PROJ), 1e6, jnp.float32)], axis=0)

    td, ti = pl.pallas_call(
        functools.partial(_knn_body, n_chunks=n_chunks, n_real=n),
        grid=(b // _BQ,),
        in_specs=[
            pl.BlockSpec((_PROJ, _BQ), lambda i: (0, i)),
            pl.BlockSpec((n_pad, _PROJ), lambda i: (0, 0)),
        ],
        out_specs=[
            pl.BlockSpec((_BQ, 16), lambda i: (i, 0)),
            pl.BlockSpec((_BQ, 16), lambda i: (i, 0)),
        ],
        out_shape=[
            jax.ShapeDtypeStruct((b, 16), jnp.float32),
            jax.ShapeDtypeStruct((b, 16), jnp.int32),
        ],
    )(qt, dpad)

    qpw = b // _NSC
    cls = pl.pallas_call(
        _cls_body,
        grid=(1,),
        in_specs=[pl.BlockSpec((n, _NCLS), lambda i: (0, 0))],
        out_specs=pl.BlockSpec((n,), lambda i: (0,)),
        out_shape=jax.ShapeDtypeStruct((n,), jnp.int32),
    )(labels)

    mesh = plsc.VectorSubcoreMesh(core_axis_name="c", subcore_axis_name="s")
    s16 = pl.kernel(
        _sc_body,
        out_type=jax.ShapeDtypeStruct((b, 16), jnp.float32),
        mesh=mesh,
        scratch_types=[
            pltpu.VMEM((qpw, 16), jnp.float32),
            pltpu.VMEM((qpw, 16), jnp.int32),
            pltpu.VMEM((n,), jnp.int32),
            pltpu.VMEM((qpw, 16), jnp.float32),
        ],
        compiler_params=pltpu.CompilerParams(needs_layout_passes=False),
    )(td, ti, cls)

    out = pl.pallas_call(
        _log_body,
        grid=(b // _BQ,),
        in_specs=[pl.BlockSpec((_BQ, 16), lambda i: (i, 0))],
        out_specs=pl.BlockSpec((_BQ, _NCLS), lambda i: (i, 0)),
        out_shape=jax.ShapeDtypeStruct((b, _NCLS), jnp.float32),
    )(s16)
    return out
